# Initial kernel scaffold; baseline (speedup 1.0000x reference)
import jax
import jax.numpy as jnp
from jax.experimental import pallas as pl


def kernel(R, Z, N, AM, NM, params):
    raise NotImplementedError("write your pallas kernel here")


# speedup vs baseline: 2.5471x; 2.5471x over previous
import jax
import jax.numpy as jnp
from jax.experimental import pallas as pl
from jax.experimental.pallas import tpu as pltpu

A = 128
NN = 48
NF = 128
RES = 20
NITER = 3
CUTOFF = 5.0
PP = 9
EDG = A * NN
CH = 16
NC = A // CH
CE = CH * NN

_F32 = jnp.float32


def _sig(x):
    return jax.nn.sigmoid(x)


def _swish(x):
    return x * _sig(x)


def _dswish(u):
    s = _sig(u)
    return s * (1.0 + u * (1.0 - s))


_PREC = jax.lax.Precision.HIGHEST


def _mm(x, w):
    return jax.lax.dot_general(x, w, (((1,), (0,)), ((), ())),
                               precision=_PREC,
                               preferred_element_type=_F32)


def _mmT(x, w):
    return jax.lax.dot_general(x, w, (((1,), (1,)), ((), ())),
                               precision=_PREC,
                               preferred_element_type=_F32)


def _scat(e1f, g):
    return jax.lax.dot_general(e1f, g, (((0,), (0,)), ((), ())),
                               precision=_PREC,
                               preferred_element_type=_F32)


def _atom_fwd(P, a, qd):
    ua = _mm(a, P['Wa1']) + P['ba1']
    am = _mm(_swish(ua), P['Wa2']) + P['ba2']
    uq = _mm(a, P['Wq1']) + P['bq1']
    q = _mm(_swish(uq), P['Wq2']) + P['bq2']
    uqm = _mm(a, P['Wqm1']) + P['bqm1']
    qm = _mm(_swish(uqm), P['Wqm2']) + P['bqm2']
    qdn = qd + q * qm
    ue = _mm(a, P['We1']) + P['be1']
    pe = _mm(_swish(ue), P['We2']) + P['be2']
    return dict(ua=ua, am=am, uq=uq, q=q, uqm=uqm, qm=qm, qdn=qdn,
                ue=ue, pe=pe)


def _chunk_fwd(P, s, t, c):
    r0 = c * CH
    e0 = c * CE
    rows = pl.ds(r0, CH)
    erows = pl.ds(e0, CE)
    ef = s['e1f'][erows, :]
    fc_c = s['fc'][rows, :]
    hr = _mm(s['rbf'][erows, :], P['Wr']) + P['br']
    rm = hr.reshape(CH, NN, NF) * fc_c[:, :, None]
    am_c = s['am'][rows, :]
    aj = _mm(ef, s['am'][:, :]).reshape(CH, NN, NF)
    ms = am_c[:, None, :] * aj * rm
    msf = ms.reshape(CE, NF)
    ub = _mm(msf, P['Wb1']) + P['bb1']
    bij = (_mm(_swish(ub), P['Wb2']) + P['bb2']).reshape(CH, NN, 1)
    ubm = _mm(msf, P['Wbm1']) + P['bbm1']
    bm = (_mm(_swish(ubm), P['Wbm2']) + P['bbm2']).reshape(CH, NN, NF)
    if t > 0:
        bdn = s['bdck'][t - 1, rows] + bij * bm
    else:
        bdn = bij * bm
    qdn_c = s['qdn'][rows, :]
    qj = _mm(ef, s['qdn'][:, :]).reshape(CH, NN, NF)
    qq = qdn_c[:, None, :] * qj
    de = jnp.sum(s['dinv'][rows, :][:, :, None] * (qq - bdn), axis=1)
    return dict(hr=hr, rm=rm, aj=aj, ub=ub, bij=bij, ubm=ubm, bm=bm,
                bdn=bdn, qj=qj, qq=qq, de=de, ef=ef, am_c=am_c,
                qdn_c=qdn_c, fc_c=fc_c, rows=rows, erows=erows)


def _mol_body(R_ref, Z_ref, N_ref, AM_ref, NM_ref, emb, pref,
              U1, c1, U2, c2, U3, c3, s):
    bnorm = jnp.sqrt(2.0 / CUTOFF)
    R = R_ref[0]
    AMc = AM_ref[0]

    def geo_body(c, _):
        rows = pl.ds(c * CH, CH)
        erows = pl.ds(c * CE, CE)
        nrows = N_ref[0, rows, :]
        nio = jax.lax.broadcasted_iota(jnp.int32, (CH, NN, A), 2)
        e1c = (nrows[:, :, None] == nio).astype(_F32).reshape(CE, A)
        s['e1f'][erows, :] = e1c
        Rj = _mm(e1c, R).reshape(CH, NN, 3)
        V = Rj - R_ref[0, rows, :][:, None, :]
        Dsq = jnp.sum(V * V, axis=2)
        Ds = jnp.sqrt(jnp.maximum(Dsq, 1e-12))
        D = jnp.where(Dsq > 1e-9, Ds, 0.0) * NM_ref[0, rows, :]
        s['dd'][rows, :] = D
        pos = D > 0
        Dsafe = jnp.where(pos, D, 1.0)
        s['dinv'][rows, :] = jnp.where(pos, 1.0 / Dsafe, 0.0)
        d = D / CUTOFF
        d2 = d * d
        d4 = d2 * d2
        d8 = d4 * d4
        d9 = d8 * d
        d10 = d9 * d
        d11 = d10 * d
        inr = d < 1.0
        s['fc'][rows, :] = jnp.where(
            inr, 1.0 - 55.0 * d9 + 99.0 * d10 - 45.0 * d11, 0.0)
        s['dfc'][rows, :] = jnp.where(
            inr, (-495.0 * d8 + 990.0 * d9 - 495.0 * d10) / CUTOFF, 0.0)
        kio = jax.lax.broadcasted_iota(jnp.int32, (CH, NN, RES),
                                       2).astype(_F32) + 1.0
        Dx = D[:, :, None]
        arg = kio * (jnp.pi / CUTOFF) * Dx
        posx = Dx > 0
        sfx = jnp.where(posx, Dx, 1.0)
        s['rbf'][erows, :] = (bnorm * jnp.where(posx, jnp.sin(arg) / sfx,
                                                0.0)).reshape(CE, RES)
        return 0

    jax.lax.fori_loop(0, NC, geo_body, 0)

    zio = jax.lax.broadcasted_iota(jnp.int32, (A, 16), 1)
    zoh = (Z_ref[0] == zio).astype(_F32)
    a0 = _mm(zoh, emb)
    s['blat'][:, :] = jnp.zeros((A, NN), _F32)

    a = a0
    qd = jnp.zeros((A, NF), _F32)
    q_lat = jnp.zeros((A, 1), _F32)
    for t in range(NITER):
        P = {k: r[t] for k, r in pref.items()}
        al = _atom_fwd(P, a, qd)
        s['ack'][t] = a
        s['qdck'][t] = qd
        s['am'][:, :] = al['am']
        s['qdn'][:, :] = al['qdn']

        def fbody(c, _, t=t, P=P):
            f = _chunk_fwd(P, s, t, c)
            if t < NITER - 1:
                s['bdck'][t, f['rows']] = f['bdn']
            s['de'][f['rows'], :] = f['de']
            s['blat'][f['rows'], :] = (s['blat'][f['rows'], :]
                                       + f['bij'][:, :, 0])
            return 0

        jax.lax.fori_loop(0, NC, fbody, 0)
        de = s['de'][:, :]
        a = a + al['pe'] * de
        qd = al['qdn']
        q_lat = q_lat + al['q']

    u1 = _mm(a, U1) + c1
    s1 = _swish(u1)
    u2 = _mm(s1, U2) + c2
    s2 = _swish(u2)
    Ei = (_mm(s2, U3) + c3) * AMc
    E = jnp.sum(Ei)

    g_s2 = _mmT(AMc, U3)
    g_s1 = _mmT(g_s2 * _dswish(u2), U2)
    ga = _mmT(g_s1 * _dswish(u1), U1)
    s['gbd'][:, :, :] = jnp.zeros((A, NN, NF), _F32)
    s['gD'][:, :] = jnp.zeros((A, NN), _F32)
    s['grbf'][:, :] = jnp.zeros((EDG, RES), _F32)

    gqd = jnp.zeros((A, NF), _F32)
    for t in range(NITER - 1, -1, -1):
        _P = {k: r[t] for k, r in pref.items()}
        P = _P
        al = _atom_fwd(P, s['ack'][t], s['qdck'][t])
        s['am'][:, :] = al['am']
        s['qdn'][:, :] = al['qdn']
        s['gde'][:, :] = ga * al['pe']
        s['gqdns'][:, :] = jnp.zeros((A, NF), _F32)
        s['gams'][:, :] = jnp.zeros((A, NF), _F32)

        def bbody(c, _, t=t, _P=_P):
            f = _chunk_fwd(_P, s, t, c)
            rows, erows = f['rows'], f['erows']
            dinv_c = s['dinv'][rows, :]
            g_de_c = s['gde'][rows, :]
            g_qq = dinv_c[:, :, None] * g_de_c[:, None, :]
            gbd_c = s['gbd'][rows] - g_qq
            s['gbd'][rows] = gbd_c
            g_Dinv = jnp.sum(g_de_c[:, None, :] * (f['qq'] - f['bdn']),
                             axis=2)
            gD_c = -(dinv_c * dinv_c) * g_Dinv
            g_bij = jnp.sum(gbd_c * f['bm'], axis=2).reshape(CE, 1)
            g_bm = (gbd_c * f['bij']).reshape(CE, NF)
            g_ms = _mmT(_mmT(g_bm, _P['Wbm2']) * _dswish(f['ubm']),
                        _P['Wbm1'])
            g_ms = g_ms + _mmT(_mmT(g_bij, _P['Wb2']) * _dswish(f['ub']),
                               _P['Wb1'])
            g_ms3 = g_ms.reshape(CH, NN, NF)
            s['gqdnr'][rows, :] = jnp.sum(g_qq * f['qj'], axis=1)
            s['gqdns'][:, :] = s['gqdns'][:, :] + _scat(
                f['ef'],
                (g_qq * f['qdn_c'][:, None, :]).reshape(CE, NF))
            am_b = f['am_c'][:, None, :]
            s['gamr'][rows, :] = jnp.sum(g_ms3 * f['aj'] * f['rm'], axis=1)
            s['gams'][:, :] = s['gams'][:, :] + _scat(
                f['ef'], (g_ms3 * am_b * f['rm']).reshape(CE, NF))
            g_rm = g_ms3 * am_b * f['aj']
            g_hr = (g_rm * f['fc_c'][:, :, None]).reshape(CE, NF)
            g_fc = jnp.sum(g_rm * f['hr'].reshape(CH, NN, NF), axis=2)
            s['gD'][rows, :] = (s['gD'][rows, :] + gD_c
                                + s['dfc'][rows, :] * g_fc)
            s['grbf'][erows, :] = (s['grbf'][erows, :]
                                   + _mmT(g_hr, _P['Wr']))
            s['de'][rows, :] = f['de']
            return 0

        jax.lax.fori_loop(0, NC, bbody, 0)
        de = s['de'][:, :]
        g_pe = ga * de
        ga = ga + _mmT(_mmT(g_pe, P['We2']) * _dswish(al['ue']), P['We1'])
        g_qdn = gqd + s['gqdnr'][:, :] + s['gqdns'][:, :]
        g_q = jnp.sum(g_qdn * al['qm'], axis=1, keepdims=True)
        g_qm = g_qdn * al['q']
        ga = ga + _mmT(_mmT(g_qm, P['Wqm2']) * _dswish(al['uqm']), P['Wqm1'])
        ga = ga + _mmT(_mmT(g_q, P['Wq2']) * _dswish(al['uq']), P['Wq1'])
        g_am = s['gamr'][:, :] + s['gams'][:, :]
        ga = ga + _mmT(_mmT(g_am, P['Wa2']) * _dswish(al['ua']), P['Wa1'])
        gqd = g_qdn

    def force_body(c, gR_sc):
        rows = pl.ds(c * CH, CH)
        erows = pl.ds(c * CE, CE)
        D = s['dd'][rows, :]
        Dx = D[:, :, None]
        posx = Dx > 0
        sfx = jnp.where(posx, Dx, 1.0)
        kio = jax.lax.broadcasted_iota(jnp.int32, (CH, NN, RES),
                                       2).astype(_F32) + 1.0
        arg = kio * (jnp.pi / CUTOFF) * Dx
        dbes = bnorm * jnp.where(
            posx,
            kio * (jnp.pi / CUTOFF) * jnp.cos(arg) / sfx
            - jnp.sin(arg) / (sfx * sfx), 0.0)
        gD = s['gD'][rows, :] + jnp.sum(
            s['grbf'][erows, :].reshape(CH, NN, RES) * dbes, axis=2)
        e1c = s['e1f'][erows, :]
        Rj = _mm(e1c, R).reshape(CH, NN, 3)
        V = Rj - R_ref[0, rows, :][:, None, :]
        Dsq = jnp.sum(V * V, axis=2)
        Ds = jnp.sqrt(jnp.maximum(Dsq, 1e-12))
        gscale = jnp.where(Dsq > 1e-9, gD * NM_ref[0, rows, :] / Ds, 0.0)
        gV = gscale[:, :, None] * V
        s['frow'][rows, :] = jnp.sum(gV, axis=1)
        return gR_sc + _scat(e1c, gV.reshape(CE, 3))

    gR_sc = jax.lax.fori_loop(0, NC, force_body, jnp.zeros((A, 3), _F32))
    F = s['frow'][:, :] - gR_sc

    Q = q_lat * AMc
    Bl = jnp.where(NM_ref[0] != 0, s['blat'][:, :], 0.0)
    return E, F, Q, Bl


_PKEYS = ['Wr', 'br', 'Wa1', 'ba1', 'Wa2', 'ba2', 'Wq1', 'bq1', 'Wq2', 'bq2',
          'Wqm1', 'bqm1', 'Wqm2', 'bqm2', 'Wb1', 'bb1', 'Wb2', 'bb2',
          'Wbm1', 'bbm1', 'Wbm2', 'bbm2', 'We1', 'be1', 'We2', 'be2']

_SCRATCH = [('e1f', (EDG, A)), ('rbf', (EDG, RES)),
            ('bdck', (NITER - 1, A, NN, NF)), ('gbd', (A, NN, NF)),
            ('grbf', (EDG, RES)), ('de', (A, NF)), ('gde', (A, NF)),
            ('am', (A, NF)), ('qdn', (A, NF)), ('gqdnr', (A, NF)),
            ('gamr', (A, NF)), ('gqdns', (A, NF)), ('gams', (A, NF)),
            ('gD', (A, NN)), ('blat', (A, NN)), ('fc', (A, NN)),
            ('dfc', (A, NN)), ('dinv', (A, NN)), ('dd', (A, NN)),
            ('frow', (A, 3)),
            ('ack', (NITER, A, NF)), ('qdck', (NITER, A, NF))]


def _body(R_ref, Z_ref, N_ref, AM_ref, NM_ref, emb_ref, *prefs):
    np_ = len(_PKEYS)
    piter_refs = prefs[:np_]
    U1_ref, c1_ref, U2_ref, c2_ref, U3_ref, c3_ref = prefs[np_:np_ + 6]
    E_ref, F_ref, Q_ref, Bl_ref = prefs[np_ + 6:np_ + 10]
    s = {k: r for (k, _), r in zip(_SCRATCH, prefs[np_ + 10:])}
    pref = dict(zip(_PKEYS, piter_refs))
    E, F, Q, Bl = _mol_body(R_ref, Z_ref, N_ref, AM_ref, NM_ref,
                            emb_ref[:, :], pref, U1_ref[:, :],
                            c1_ref[:, :], U2_ref[:, :], c2_ref[:, :],
                            U3_ref[:, :], c3_ref[:, :], s)
    E_ref[0] = jnp.zeros((1, 128), _F32) + E
    F_ref[0] = F
    Q_ref[0] = Q
    Bl_ref[0] = Bl


def _pack(params):
    nm_map = [('rbf', 0, 'Wr', 'br'), ('phi_a', 0, 'Wa1', 'ba1'),
              ('phi_a', 1, 'Wa2', 'ba2'), ('phi_q', 0, 'Wq1', 'bq1'),
              ('phi_q', 1, 'Wq2', 'bq2'), ('phi_qm', 0, 'Wqm1', 'bqm1'),
              ('phi_qm', 1, 'Wqm2', 'bqm2'), ('phi_b', 0, 'Wb1', 'bb1'),
              ('phi_b', 1, 'Wb2', 'bb2'), ('phi_bm', 0, 'Wbm1', 'bbm1'),
              ('phi_bm', 1, 'Wbm2', 'bbm2'), ('phi_e', 0, 'We1', 'be1'),
              ('phi_e', 1, 'We2', 'be2')]
    out = {}
    for name, li, wk, bk in nm_map:
        ws, bs = [], []
        for t in range(NITER):
            p = params['iters'][t][name]
            if isinstance(p, list):
                p = p[li]
            ws.append(p['w'])
            bs.append(p['b'].reshape(1, -1))
        out[wk] = jnp.stack(ws)
        out[bk] = jnp.stack(bs)
    emb = params['atom_emb']
    emb16 = jnp.zeros((16, NF), _F32).at[:emb.shape[0]].set(emb)
    at = params['atomic']
    return (out, emb16, at[0]['w'], at[0]['b'].reshape(1, -1),
            at[1]['w'], at[1]['b'].reshape(1, -1),
            at[2]['w'], at[2]['b'].reshape(1, -1))


def kernel(R, Z, N, AM, NM, params):
    B = R.shape[0]
    pk, emb16, U1, c1, U2, c2, U3, c3 = _pack(params)
    Zc = Z.astype(jnp.int32).reshape(B, A, 1)
    Nc = N.astype(jnp.int32)
    AMc = AM.reshape(B, A, 1)

    def cspec(x):
        nd = x.ndim
        return pl.BlockSpec(x.shape, lambda b, _n=nd: (0,) * _n)

    in_specs = [
        pl.BlockSpec((1, A, 3), lambda b: (b, 0, 0)),
        pl.BlockSpec((1, A, 1), lambda b: (b, 0, 0)),
        pl.BlockSpec((1, A, NN), lambda b: (b, 0, 0)),
        pl.BlockSpec((1, A, 1), lambda b: (b, 0, 0)),
        pl.BlockSpec((1, A, NN), lambda b: (b, 0, 0)),
        cspec(emb16),
    ]
    pargs = []
    for k in _PKEYS:
        pargs.append(pk[k])
        in_specs.append(cspec(pk[k]))
    for arr in (U1, c1, U2, c2, U3, c3):
        pargs.append(arr)
        in_specs.append(cspec(arr))

    out_specs = [
        pl.BlockSpec((1, 1, 128), lambda b: (b, 0, 0)),
        pl.BlockSpec((1, A, 3), lambda b: (b, 0, 0)),
        pl.BlockSpec((1, A, 1), lambda b: (b, 0, 0)),
        pl.BlockSpec((1, A, NN), lambda b: (b, 0, 0)),
    ]
    out_shape = [
        jax.ShapeDtypeStruct((B, 1, 128), _F32),
        jax.ShapeDtypeStruct((B, A, 3), _F32),
        jax.ShapeDtypeStruct((B, A, 1), _F32),
        jax.ShapeDtypeStruct((B, A, NN), _F32),
    ]
    Eo, Fo, Qo, Blo = pl.pallas_call(
        _body,
        grid=(B,),
        in_specs=in_specs,
        out_specs=out_specs,
        out_shape=out_shape,
        scratch_shapes=[pltpu.VMEM(shape, _F32) for _, shape in _SCRATCH],
        compiler_params=pltpu.CompilerParams(
            dimension_semantics=("arbitrary",),
        ),
    )(R, Zc, Nc, AMc, NM, emb16, *pargs)
    return (Eo[:, 0, :1], Fo, Qo[:, :, 0], Blo)


# speedup vs baseline: 2.6709x; 1.0486x over previous
import jax
import jax.numpy as jnp
from jax.experimental import pallas as pl
from jax.experimental.pallas import tpu as pltpu

A = 128
NN = 48
NF = 128
RES = 20
NITER = 3
CUTOFF = 5.0
PP = 9
EDG = A * NN
CH = 32
NC = A // CH
CE = CH * NN

_F32 = jnp.float32


def _sig(x):
    return jax.nn.sigmoid(x)


def _swish(x):
    return x * _sig(x)


def _dswish(u):
    s = _sig(u)
    return s * (1.0 + u * (1.0 - s))


_PREC = jax.lax.Precision.HIGHEST


def _mm(x, w):
    return jax.lax.dot_general(x, w, (((1,), (0,)), ((), ())),
                               precision=_PREC,
                               preferred_element_type=_F32)


def _mmT(x, w):
    return jax.lax.dot_general(x, w, (((1,), (1,)), ((), ())),
                               precision=_PREC,
                               preferred_element_type=_F32)


def _scat(e1f, g):
    return jax.lax.dot_general(e1f, g, (((0,), (0,)), ((), ())),
                               precision=_PREC,
                               preferred_element_type=_F32)


def _atom_fwd(P, a, qd):
    ua = _mm(a, P['Wa1']) + P['ba1']
    am = _mm(_swish(ua), P['Wa2']) + P['ba2']
    uq = _mm(a, P['Wq1']) + P['bq1']
    q = _mm(_swish(uq), P['Wq2']) + P['bq2']
    uqm = _mm(a, P['Wqm1']) + P['bqm1']
    qm = _mm(_swish(uqm), P['Wqm2']) + P['bqm2']
    qdn = qd + q * qm
    ue = _mm(a, P['We1']) + P['be1']
    pe = _mm(_swish(ue), P['We2']) + P['be2']
    return dict(ua=ua, am=am, uq=uq, q=q, uqm=uqm, qm=qm, qdn=qdn,
                ue=ue, pe=pe)


def _chunk_fwd(P, s, t, c):
    r0 = c * CH
    e0 = c * CE
    rows = pl.ds(r0, CH)
    erows = pl.ds(e0, CE)
    ef = s['e1f'][erows, :]
    fc_c = s['fc'][rows, :]
    hr = _mm(s['rbf'][erows, :], P['Wr']) + P['br']
    rm = hr.reshape(CH, NN, NF) * fc_c[:, :, None]
    am_c = s['am'][rows, :]
    aj = _mm(ef, s['am'][:, :]).reshape(CH, NN, NF)
    ms = am_c[:, None, :] * aj * rm
    msf = ms.reshape(CE, NF)
    ub = _mm(msf, P['Wb1']) + P['bb1']
    bij = (_mm(_swish(ub), P['Wb2']) + P['bb2']).reshape(CH, NN, 1)
    ubm = _mm(msf, P['Wbm1']) + P['bbm1']
    bm = (_mm(_swish(ubm), P['Wbm2']) + P['bbm2']).reshape(CH, NN, NF)
    if t > 0:
        bdn = s['bdck'][t - 1, rows] + bij * bm
    else:
        bdn = bij * bm
    qdn_c = s['qdn'][rows, :]
    qj = _mm(ef, s['qdn'][:, :]).reshape(CH, NN, NF)
    qq = qdn_c[:, None, :] * qj
    de = jnp.sum(s['dinv'][rows, :][:, :, None] * (qq - bdn), axis=1)
    return dict(hr=hr, rm=rm, aj=aj, ub=ub, bij=bij, ubm=ubm, bm=bm,
                bdn=bdn, qj=qj, qq=qq, de=de, ef=ef, am_c=am_c,
                qdn_c=qdn_c, fc_c=fc_c, rows=rows, erows=erows)


def _mol_body(R_ref, Z_ref, N_ref, AM_ref, NM_ref, emb, pref,
              U1, c1, U2, c2, U3, c3, s):
    bnorm = jnp.sqrt(2.0 / CUTOFF)
    R = R_ref[0]
    AMc = AM_ref[0]

    def geo_body(c, _):
        rows = pl.ds(c * CH, CH)
        erows = pl.ds(c * CE, CE)
        nrows = N_ref[0, rows, :]
        nio = jax.lax.broadcasted_iota(jnp.int32, (CH, NN, A), 2)
        e1c = (nrows[:, :, None] == nio).astype(_F32).reshape(CE, A)
        s['e1f'][erows, :] = e1c
        Rj = _mm(e1c, R).reshape(CH, NN, 3)
        V = Rj - R_ref[0, rows, :][:, None, :]
        Dsq = jnp.sum(V * V, axis=2)
        Ds = jnp.sqrt(jnp.maximum(Dsq, 1e-12))
        D = jnp.where(Dsq > 1e-9, Ds, 0.0) * NM_ref[0, rows, :]
        s['dd'][rows, :] = D
        pos = D > 0
        Dsafe = jnp.where(pos, D, 1.0)
        s['dinv'][rows, :] = jnp.where(pos, 1.0 / Dsafe, 0.0)
        d = D / CUTOFF
        d2 = d * d
        d4 = d2 * d2
        d8 = d4 * d4
        d9 = d8 * d
        d10 = d9 * d
        d11 = d10 * d
        inr = d < 1.0
        s['fc'][rows, :] = jnp.where(
            inr, 1.0 - 55.0 * d9 + 99.0 * d10 - 45.0 * d11, 0.0)
        s['dfc'][rows, :] = jnp.where(
            inr, (-495.0 * d8 + 990.0 * d9 - 495.0 * d10) / CUTOFF, 0.0)
        kio = jax.lax.broadcasted_iota(jnp.int32, (CH, NN, RES),
                                       2).astype(_F32) + 1.0
        Dx = D[:, :, None]
        arg = kio * (jnp.pi / CUTOFF) * Dx
        posx = Dx > 0
        sfx = jnp.where(posx, Dx, 1.0)
        s['rbf'][erows, :] = (bnorm * jnp.where(posx, jnp.sin(arg) / sfx,
                                                0.0)).reshape(CE, RES)
        return 0

    jax.lax.fori_loop(0, NC, geo_body, 0)

    zio = jax.lax.broadcasted_iota(jnp.int32, (A, 16), 1)
    zoh = (Z_ref[0] == zio).astype(_F32)
    a0 = _mm(zoh, emb)
    s['blat'][:, :] = jnp.zeros((A, NN), _F32)

    a = a0
    qd = jnp.zeros((A, NF), _F32)
    q_lat = jnp.zeros((A, 1), _F32)
    for t in range(NITER):
        P = {k: r[t] for k, r in pref.items()}
        al = _atom_fwd(P, a, qd)
        s['ack'][t] = a
        s['qdck'][t] = qd
        s['am'][:, :] = al['am']
        s['qdn'][:, :] = al['qdn']

        def fbody(c, _, t=t, P=P):
            f = _chunk_fwd(P, s, t, c)
            if t < NITER - 1:
                s['bdck'][t, f['rows']] = f['bdn']
            s['de'][f['rows'], :] = f['de']
            s['blat'][f['rows'], :] = (s['blat'][f['rows'], :]
                                       + f['bij'][:, :, 0])
            return 0

        jax.lax.fori_loop(0, NC, fbody, 0)
        de = s['de'][:, :]
        a = a + al['pe'] * de
        qd = al['qdn']
        q_lat = q_lat + al['q']

    u1 = _mm(a, U1) + c1
    s1 = _swish(u1)
    u2 = _mm(s1, U2) + c2
    s2 = _swish(u2)
    Ei = (_mm(s2, U3) + c3) * AMc
    E = jnp.sum(Ei)

    g_s2 = _mmT(AMc, U3)
    g_s1 = _mmT(g_s2 * _dswish(u2), U2)
    ga = _mmT(g_s1 * _dswish(u1), U1)
    s['gbd'][:, :, :] = jnp.zeros((A, NN, NF), _F32)
    s['gD'][:, :] = jnp.zeros((A, NN), _F32)
    s['grbf'][:, :] = jnp.zeros((EDG, RES), _F32)

    gqd = jnp.zeros((A, NF), _F32)
    for t in range(NITER - 1, -1, -1):
        _P = {k: r[t] for k, r in pref.items()}
        P = _P
        al = _atom_fwd(P, s['ack'][t], s['qdck'][t])
        s['am'][:, :] = al['am']
        s['qdn'][:, :] = al['qdn']
        s['gde'][:, :] = ga * al['pe']
        s['gqdns'][:, :] = jnp.zeros((A, NF), _F32)
        s['gams'][:, :] = jnp.zeros((A, NF), _F32)

        def bbody(c, _, t=t, _P=_P):
            f = _chunk_fwd(_P, s, t, c)
            rows, erows = f['rows'], f['erows']
            dinv_c = s['dinv'][rows, :]
            g_de_c = s['gde'][rows, :]
            g_qq = dinv_c[:, :, None] * g_de_c[:, None, :]
            gbd_c = s['gbd'][rows] - g_qq
            s['gbd'][rows] = gbd_c
            g_Dinv = jnp.sum(g_de_c[:, None, :] * (f['qq'] - f['bdn']),
                             axis=2)
            gD_c = -(dinv_c * dinv_c) * g_Dinv
            g_bij = jnp.sum(gbd_c * f['bm'], axis=2).reshape(CE, 1)
            g_bm = (gbd_c * f['bij']).reshape(CE, NF)
            g_ms = _mmT(_mmT(g_bm, _P['Wbm2']) * _dswish(f['ubm']),
                        _P['Wbm1'])
            g_ms = g_ms + _mmT(_mmT(g_bij, _P['Wb2']) * _dswish(f['ub']),
                               _P['Wb1'])
            g_ms3 = g_ms.reshape(CH, NN, NF)
            s['gqdnr'][rows, :] = jnp.sum(g_qq * f['qj'], axis=1)
            s['gqdns'][:, :] = s['gqdns'][:, :] + _scat(
                f['ef'],
                (g_qq * f['qdn_c'][:, None, :]).reshape(CE, NF))
            am_b = f['am_c'][:, None, :]
            s['gamr'][rows, :] = jnp.sum(g_ms3 * f['aj'] * f['rm'], axis=1)
            s['gams'][:, :] = s['gams'][:, :] + _scat(
                f['ef'], (g_ms3 * am_b * f['rm']).reshape(CE, NF))
            g_rm = g_ms3 * am_b * f['aj']
            g_hr = (g_rm * f['fc_c'][:, :, None]).reshape(CE, NF)
            g_fc = jnp.sum(g_rm * f['hr'].reshape(CH, NN, NF), axis=2)
            s['gD'][rows, :] = (s['gD'][rows, :] + gD_c
                                + s['dfc'][rows, :] * g_fc)
            s['grbf'][erows, :] = (s['grbf'][erows, :]
                                   + _mmT(g_hr, _P['Wr']))
            s['de'][rows, :] = f['de']
            return 0

        jax.lax.fori_loop(0, NC, bbody, 0)
        de = s['de'][:, :]
        g_pe = ga * de
        ga = ga + _mmT(_mmT(g_pe, P['We2']) * _dswish(al['ue']), P['We1'])
        g_qdn = gqd + s['gqdnr'][:, :] + s['gqdns'][:, :]
        g_q = jnp.sum(g_qdn * al['qm'], axis=1, keepdims=True)
        g_qm = g_qdn * al['q']
        ga = ga + _mmT(_mmT(g_qm, P['Wqm2']) * _dswish(al['uqm']), P['Wqm1'])
        ga = ga + _mmT(_mmT(g_q, P['Wq2']) * _dswish(al['uq']), P['Wq1'])
        g_am = s['gamr'][:, :] + s['gams'][:, :]
        ga = ga + _mmT(_mmT(g_am, P['Wa2']) * _dswish(al['ua']), P['Wa1'])
        gqd = g_qdn

    def force_body(c, gR_sc):
        rows = pl.ds(c * CH, CH)
        erows = pl.ds(c * CE, CE)
        D = s['dd'][rows, :]
        Dx = D[:, :, None]
        posx = Dx > 0
        sfx = jnp.where(posx, Dx, 1.0)
        kio = jax.lax.broadcasted_iota(jnp.int32, (CH, NN, RES),
                                       2).astype(_F32) + 1.0
        arg = kio * (jnp.pi / CUTOFF) * Dx
        dbes = bnorm * jnp.where(
            posx,
            kio * (jnp.pi / CUTOFF) * jnp.cos(arg) / sfx
            - jnp.sin(arg) / (sfx * sfx), 0.0)
        gD = s['gD'][rows, :] + jnp.sum(
            s['grbf'][erows, :].reshape(CH, NN, RES) * dbes, axis=2)
        e1c = s['e1f'][erows, :]
        Rj = _mm(e1c, R).reshape(CH, NN, 3)
        V = Rj - R_ref[0, rows, :][:, None, :]
        Dsq = jnp.sum(V * V, axis=2)
        Ds = jnp.sqrt(jnp.maximum(Dsq, 1e-12))
        gscale = jnp.where(Dsq > 1e-9, gD * NM_ref[0, rows, :] / Ds, 0.0)
        gV = gscale[:, :, None] * V
        s['frow'][rows, :] = jnp.sum(gV, axis=1)
        return gR_sc + _scat(e1c, gV.reshape(CE, 3))

    gR_sc = jax.lax.fori_loop(0, NC, force_body, jnp.zeros((A, 3), _F32))
    F = s['frow'][:, :] - gR_sc

    Q = q_lat * AMc
    Bl = jnp.where(NM_ref[0] != 0, s['blat'][:, :], 0.0)
    return E, F, Q, Bl


_PKEYS = ['Wr', 'br', 'Wa1', 'ba1', 'Wa2', 'ba2', 'Wq1', 'bq1', 'Wq2', 'bq2',
          'Wqm1', 'bqm1', 'Wqm2', 'bqm2', 'Wb1', 'bb1', 'Wb2', 'bb2',
          'Wbm1', 'bbm1', 'Wbm2', 'bbm2', 'We1', 'be1', 'We2', 'be2']

_SCRATCH = [('e1f', (EDG, A)), ('rbf', (EDG, RES)),
            ('bdck', (NITER - 1, A, NN, NF)), ('gbd', (A, NN, NF)),
            ('grbf', (EDG, RES)), ('de', (A, NF)), ('gde', (A, NF)),
            ('am', (A, NF)), ('qdn', (A, NF)), ('gqdnr', (A, NF)),
            ('gamr', (A, NF)), ('gqdns', (A, NF)), ('gams', (A, NF)),
            ('gD', (A, NN)), ('blat', (A, NN)), ('fc', (A, NN)),
            ('dfc', (A, NN)), ('dinv', (A, NN)), ('dd', (A, NN)),
            ('frow', (A, 3)),
            ('ack', (NITER, A, NF)), ('qdck', (NITER, A, NF))]


def _body(R_ref, Z_ref, N_ref, AM_ref, NM_ref, emb_ref, *prefs):
    np_ = len(_PKEYS)
    piter_refs = prefs[:np_]
    U1_ref, c1_ref, U2_ref, c2_ref, U3_ref, c3_ref = prefs[np_:np_ + 6]
    E_ref, F_ref, Q_ref, Bl_ref = prefs[np_ + 6:np_ + 10]
    s = {k: r for (k, _), r in zip(_SCRATCH, prefs[np_ + 10:])}
    pref = dict(zip(_PKEYS, piter_refs))
    E, F, Q, Bl = _mol_body(R_ref, Z_ref, N_ref, AM_ref, NM_ref,
                            emb_ref[:, :], pref, U1_ref[:, :],
                            c1_ref[:, :], U2_ref[:, :], c2_ref[:, :],
                            U3_ref[:, :], c3_ref[:, :], s)
    E_ref[0] = jnp.zeros((1, 128), _F32) + E
    F_ref[0] = F
    Q_ref[0] = Q
    Bl_ref[0] = Bl


def _pack(params):
    nm_map = [('rbf', 0, 'Wr', 'br'), ('phi_a', 0, 'Wa1', 'ba1'),
              ('phi_a', 1, 'Wa2', 'ba2'), ('phi_q', 0, 'Wq1', 'bq1'),
              ('phi_q', 1, 'Wq2', 'bq2'), ('phi_qm', 0, 'Wqm1', 'bqm1'),
              ('phi_qm', 1, 'Wqm2', 'bqm2'), ('phi_b', 0, 'Wb1', 'bb1'),
              ('phi_b', 1, 'Wb2', 'bb2'), ('phi_bm', 0, 'Wbm1', 'bbm1'),
              ('phi_bm', 1, 'Wbm2', 'bbm2'), ('phi_e', 0, 'We1', 'be1'),
              ('phi_e', 1, 'We2', 'be2')]
    out = {}
    for name, li, wk, bk in nm_map:
        ws, bs = [], []
        for t in range(NITER):
            p = params['iters'][t][name]
            if isinstance(p, list):
                p = p[li]
            ws.append(p['w'])
            bs.append(p['b'].reshape(1, -1))
        out[wk] = jnp.stack(ws)
        out[bk] = jnp.stack(bs)
    emb = params['atom_emb']
    emb16 = jnp.zeros((16, NF), _F32).at[:emb.shape[0]].set(emb)
    at = params['atomic']
    return (out, emb16, at[0]['w'], at[0]['b'].reshape(1, -1),
            at[1]['w'], at[1]['b'].reshape(1, -1),
            at[2]['w'], at[2]['b'].reshape(1, -1))


def kernel(R, Z, N, AM, NM, params):
    B = R.shape[0]
    pk, emb16, U1, c1, U2, c2, U3, c3 = _pack(params)
    Zc = Z.astype(jnp.int32).reshape(B, A, 1)
    Nc = N.astype(jnp.int32)
    AMc = AM.reshape(B, A, 1)

    def cspec(x):
        nd = x.ndim
        return pl.BlockSpec(x.shape, lambda b, _n=nd: (0,) * _n)

    in_specs = [
        pl.BlockSpec((1, A, 3), lambda b: (b, 0, 0)),
        pl.BlockSpec((1, A, 1), lambda b: (b, 0, 0)),
        pl.BlockSpec((1, A, NN), lambda b: (b, 0, 0)),
        pl.BlockSpec((1, A, 1), lambda b: (b, 0, 0)),
        pl.BlockSpec((1, A, NN), lambda b: (b, 0, 0)),
        cspec(emb16),
    ]
    pargs = []
    for k in _PKEYS:
        pargs.append(pk[k])
        in_specs.append(cspec(pk[k]))
    for arr in (U1, c1, U2, c2, U3, c3):
        pargs.append(arr)
        in_specs.append(cspec(arr))

    out_specs = [
        pl.BlockSpec((1, 1, 128), lambda b: (b, 0, 0)),
        pl.BlockSpec((1, A, 3), lambda b: (b, 0, 0)),
        pl.BlockSpec((1, A, 1), lambda b: (b, 0, 0)),
        pl.BlockSpec((1, A, NN), lambda b: (b, 0, 0)),
    ]
    out_shape = [
        jax.ShapeDtypeStruct((B, 1, 128), _F32),
        jax.ShapeDtypeStruct((B, A, 3), _F32),
        jax.ShapeDtypeStruct((B, A, 1), _F32),
        jax.ShapeDtypeStruct((B, A, NN), _F32),
    ]
    Eo, Fo, Qo, Blo = pl.pallas_call(
        _body,
        grid=(B,),
        in_specs=in_specs,
        out_specs=out_specs,
        out_shape=out_shape,
        scratch_shapes=[pltpu.VMEM(shape, _F32) for _, shape in _SCRATCH],
        compiler_params=pltpu.CompilerParams(
            dimension_semantics=("arbitrary",),
        ),
    )(R, Zc, Nc, AMc, NM, emb16, *pargs)
    return (Eo[:, 0, :1], Fo, Qo[:, :, 0], Blo)


# speedup vs baseline: 4.4664x; 1.6723x over previous
import jax
import jax.numpy as jnp
from jax.experimental import pallas as pl
from jax.experimental.pallas import tpu as pltpu

A = 128
NN = 48
NF = 128
RES = 20
NITER = 3
CUTOFF = 5.0
PP = 9
EDG = A * NN
CH = 32
NC = A // CH
CE = CH * NN

_F32 = jnp.float32


def _sig(x):
    return jax.nn.sigmoid(x)


def _swish(x):
    return x * _sig(x)


def _dswish(u):
    s = _sig(u)
    return s * (1.0 + u * (1.0 - s))


_PREC = jax.lax.Precision.HIGHEST
_BF16 = jnp.bfloat16


def _mm(x, w):
    return jax.lax.dot_general(x.astype(_BF16), w.astype(_BF16),
                               (((1,), (0,)), ((), ())),
                               preferred_element_type=_F32)


def _mmT(x, w):
    return jax.lax.dot_general(x.astype(_BF16), w.astype(_BF16),
                               (((1,), (1,)), ((), ())),
                               preferred_element_type=_F32)


def _mmx(x, w):
    return jax.lax.dot_general(x, w, (((1,), (0,)), ((), ())),
                               precision=_PREC,
                               preferred_element_type=_F32)


def _scat(e1f, g):
    return jax.lax.dot_general(e1f, g, (((0,), (0,)), ((), ())),
                               precision=_PREC,
                               preferred_element_type=_F32)


def _atom_fwd(P, a, qd):
    ua = _mm(a, P['Wa1']) + P['ba1']
    am = _mm(_swish(ua), P['Wa2']) + P['ba2']
    uq = _mm(a, P['Wq1']) + P['bq1']
    q = _mm(_swish(uq), P['Wq2']) + P['bq2']
    uqm = _mm(a, P['Wqm1']) + P['bqm1']
    qm = _mm(_swish(uqm), P['Wqm2']) + P['bqm2']
    qdn = qd + q * qm
    ue = _mm(a, P['We1']) + P['be1']
    pe = _mm(_swish(ue), P['We2']) + P['be2']
    return dict(ua=ua, am=am, uq=uq, q=q, uqm=uqm, qm=qm, qdn=qdn,
                ue=ue, pe=pe)


def _chunk_fwd(P, s, t, c):
    r0 = c * CH
    e0 = c * CE
    rows = pl.ds(r0, CH)
    erows = pl.ds(e0, CE)
    ef = s['e1f'][erows, :]
    fc_c = s['fc'][rows, :]
    hr = _mm(s['rbf'][erows, :], P['Wr']) + P['br']
    rm = hr.reshape(CH, NN, NF) * fc_c[:, :, None]
    am_c = s['am'][rows, :]
    aj = _mmx(ef, s['am'][:, :]).reshape(CH, NN, NF)
    ms = am_c[:, None, :] * aj * rm
    msf = ms.reshape(CE, NF)
    ub = _mm(msf, P['Wb1']) + P['bb1']
    bij = (_mm(_swish(ub), P['Wb2']) + P['bb2']).reshape(CH, NN, 1)
    ubm = _mm(msf, P['Wbm1']) + P['bbm1']
    bm = (_mm(_swish(ubm), P['Wbm2']) + P['bbm2']).reshape(CH, NN, NF)
    if t > 0:
        bdn = s['bdck'][t - 1, rows] + bij * bm
    else:
        bdn = bij * bm
    qdn_c = s['qdn'][rows, :]
    qj = _mmx(ef, s['qdn'][:, :]).reshape(CH, NN, NF)
    qq = qdn_c[:, None, :] * qj
    de = jnp.sum(s['dinv'][rows, :][:, :, None] * (qq - bdn), axis=1)
    return dict(hr=hr, rm=rm, aj=aj, ub=ub, bij=bij, ubm=ubm, bm=bm,
                bdn=bdn, qj=qj, qq=qq, de=de, ef=ef, am_c=am_c,
                qdn_c=qdn_c, fc_c=fc_c, rows=rows, erows=erows)


def _mol_body(R_ref, Z_ref, N_ref, AM_ref, NM_ref, emb, pref,
              U1, c1, U2, c2, U3, c3, s):
    bnorm = jnp.sqrt(2.0 / CUTOFF)
    R = R_ref[0]
    AMc = AM_ref[0]

    def geo_body(c, _):
        rows = pl.ds(c * CH, CH)
        erows = pl.ds(c * CE, CE)
        nrows = N_ref[0, rows, :]
        nio = jax.lax.broadcasted_iota(jnp.int32, (CH, NN, A), 2)
        e1c = (nrows[:, :, None] == nio).astype(_F32).reshape(CE, A)
        s['e1f'][erows, :] = e1c
        Rj = _mmx(e1c, R).reshape(CH, NN, 3)
        V = Rj - R_ref[0, rows, :][:, None, :]
        Dsq = jnp.sum(V * V, axis=2)
        Ds = jnp.sqrt(jnp.maximum(Dsq, 1e-12))
        D = jnp.where(Dsq > 1e-9, Ds, 0.0) * NM_ref[0, rows, :]
        s['dd'][rows, :] = D
        pos = D > 0
        Dsafe = jnp.where(pos, D, 1.0)
        s['dinv'][rows, :] = jnp.where(pos, 1.0 / Dsafe, 0.0)
        d = D / CUTOFF
        d2 = d * d
        d4 = d2 * d2
        d8 = d4 * d4
        d9 = d8 * d
        d10 = d9 * d
        d11 = d10 * d
        inr = d < 1.0
        s['fc'][rows, :] = jnp.where(
            inr, 1.0 - 55.0 * d9 + 99.0 * d10 - 45.0 * d11, 0.0)
        s['dfc'][rows, :] = jnp.where(
            inr, (-495.0 * d8 + 990.0 * d9 - 495.0 * d10) / CUTOFF, 0.0)
        kio = jax.lax.broadcasted_iota(jnp.int32, (CH, NN, RES),
                                       2).astype(_F32) + 1.0
        Dx = D[:, :, None]
        arg = kio * (jnp.pi / CUTOFF) * Dx
        posx = Dx > 0
        sfx = jnp.where(posx, Dx, 1.0)
        s['rbf'][erows, :] = (bnorm * jnp.where(posx, jnp.sin(arg) / sfx,
                                                0.0)).reshape(CE, RES)
        return 0

    jax.lax.fori_loop(0, NC, geo_body, 0)

    zio = jax.lax.broadcasted_iota(jnp.int32, (A, 16), 1)
    zoh = (Z_ref[0] == zio).astype(_F32)
    a0 = _mmx(zoh, emb)
    s['blat'][:, :] = jnp.zeros((A, NN), _F32)

    a = a0
    qd = jnp.zeros((A, NF), _F32)
    q_lat = jnp.zeros((A, 1), _F32)
    for t in range(NITER):
        P = {k: r[t] for k, r in pref.items()}
        al = _atom_fwd(P, a, qd)
        s['ack'][t] = a
        s['qdck'][t] = qd
        s['am'][:, :] = al['am']
        s['qdn'][:, :] = al['qdn']

        def fbody(c, _, t=t, P=P):
            f = _chunk_fwd(P, s, t, c)
            if t < NITER - 1:
                s['bdck'][t, f['rows']] = f['bdn']
            s['de'][f['rows'], :] = f['de']
            s['blat'][f['rows'], :] = (s['blat'][f['rows'], :]
                                       + f['bij'][:, :, 0])
            return 0

        jax.lax.fori_loop(0, NC, fbody, 0)
        de = s['de'][:, :]
        a = a + al['pe'] * de
        qd = al['qdn']
        q_lat = q_lat + al['q']

    u1 = _mm(a, U1) + c1
    s1 = _swish(u1)
    u2 = _mm(s1, U2) + c2
    s2 = _swish(u2)
    Ei = (_mm(s2, U3) + c3) * AMc
    E = jnp.sum(Ei)

    g_s2 = _mmT(AMc, U3)
    g_s1 = _mmT(g_s2 * _dswish(u2), U2)
    ga = _mmT(g_s1 * _dswish(u1), U1)
    s['gbd'][:, :, :] = jnp.zeros((A, NN, NF), _F32)
    s['gD'][:, :] = jnp.zeros((A, NN), _F32)
    s['grbf'][:, :] = jnp.zeros((EDG, RES), _F32)

    gqd = jnp.zeros((A, NF), _F32)
    for t in range(NITER - 1, -1, -1):
        _P = {k: r[t] for k, r in pref.items()}
        P = _P
        al = _atom_fwd(P, s['ack'][t], s['qdck'][t])
        s['am'][:, :] = al['am']
        s['qdn'][:, :] = al['qdn']
        s['gde'][:, :] = ga * al['pe']
        s['gqdns'][:, :] = jnp.zeros((A, NF), _F32)
        s['gams'][:, :] = jnp.zeros((A, NF), _F32)

        def bbody(c, _, t=t, _P=_P):
            f = _chunk_fwd(_P, s, t, c)
            rows, erows = f['rows'], f['erows']
            dinv_c = s['dinv'][rows, :]
            g_de_c = s['gde'][rows, :]
            g_qq = dinv_c[:, :, None] * g_de_c[:, None, :]
            gbd_c = s['gbd'][rows] - g_qq
            s['gbd'][rows] = gbd_c
            g_Dinv = jnp.sum(g_de_c[:, None, :] * (f['qq'] - f['bdn']),
                             axis=2)
            gD_c = -(dinv_c * dinv_c) * g_Dinv
            g_bij = jnp.sum(gbd_c * f['bm'], axis=2).reshape(CE, 1)
            g_bm = (gbd_c * f['bij']).reshape(CE, NF)
            g_ms = _mmT(_mmT(g_bm, _P['Wbm2']) * _dswish(f['ubm']),
                        _P['Wbm1'])
            g_ms = g_ms + _mmT(_mmT(g_bij, _P['Wb2']) * _dswish(f['ub']),
                               _P['Wb1'])
            g_ms3 = g_ms.reshape(CH, NN, NF)
            s['gqdnr'][rows, :] = jnp.sum(g_qq * f['qj'], axis=1)
            s['gqdns'][:, :] = s['gqdns'][:, :] + _scat(
                f['ef'],
                (g_qq * f['qdn_c'][:, None, :]).reshape(CE, NF))
            am_b = f['am_c'][:, None, :]
            s['gamr'][rows, :] = jnp.sum(g_ms3 * f['aj'] * f['rm'], axis=1)
            s['gams'][:, :] = s['gams'][:, :] + _scat(
                f['ef'], (g_ms3 * am_b * f['rm']).reshape(CE, NF))
            g_rm = g_ms3 * am_b * f['aj']
            g_hr = (g_rm * f['fc_c'][:, :, None]).reshape(CE, NF)
            g_fc = jnp.sum(g_rm * f['hr'].reshape(CH, NN, NF), axis=2)
            s['gD'][rows, :] = (s['gD'][rows, :] + gD_c
                                + s['dfc'][rows, :] * g_fc)
            s['grbf'][erows, :] = (s['grbf'][erows, :]
                                   + _mmT(g_hr, _P['Wr']))
            s['de'][rows, :] = f['de']
            return 0

        jax.lax.fori_loop(0, NC, bbody, 0)
        de = s['de'][:, :]
        g_pe = ga * de
        ga = ga + _mmT(_mmT(g_pe, P['We2']) * _dswish(al['ue']), P['We1'])
        g_qdn = gqd + s['gqdnr'][:, :] + s['gqdns'][:, :]
        g_q = jnp.sum(g_qdn * al['qm'], axis=1, keepdims=True)
        g_qm = g_qdn * al['q']
        ga = ga + _mmT(_mmT(g_qm, P['Wqm2']) * _dswish(al['uqm']), P['Wqm1'])
        ga = ga + _mmT(_mmT(g_q, P['Wq2']) * _dswish(al['uq']), P['Wq1'])
        g_am = s['gamr'][:, :] + s['gams'][:, :]
        ga = ga + _mmT(_mmT(g_am, P['Wa2']) * _dswish(al['ua']), P['Wa1'])
        gqd = g_qdn

    def force_body(c, gR_sc):
        rows = pl.ds(c * CH, CH)
        erows = pl.ds(c * CE, CE)
        D = s['dd'][rows, :]
        Dx = D[:, :, None]
        posx = Dx > 0
        sfx = jnp.where(posx, Dx, 1.0)
        kio = jax.lax.broadcasted_iota(jnp.int32, (CH, NN, RES),
                                       2).astype(_F32) + 1.0
        arg = kio * (jnp.pi / CUTOFF) * Dx
        dbes = bnorm * jnp.where(
            posx,
            kio * (jnp.pi / CUTOFF) * jnp.cos(arg) / sfx
            - jnp.sin(arg) / (sfx * sfx), 0.0)
        gD = s['gD'][rows, :] + jnp.sum(
            s['grbf'][erows, :].reshape(CH, NN, RES) * dbes, axis=2)
        e1c = s['e1f'][erows, :]
        Rj = _mmx(e1c, R).reshape(CH, NN, 3)
        V = Rj - R_ref[0, rows, :][:, None, :]
        Dsq = jnp.sum(V * V, axis=2)
        Ds = jnp.sqrt(jnp.maximum(Dsq, 1e-12))
        gscale = jnp.where(Dsq > 1e-9, gD * NM_ref[0, rows, :] / Ds, 0.0)
        gV = gscale[:, :, None] * V
        s['frow'][rows, :] = jnp.sum(gV, axis=1)
        return gR_sc + _scat(e1c, gV.reshape(CE, 3))

    gR_sc = jax.lax.fori_loop(0, NC, force_body, jnp.zeros((A, 3), _F32))
    F = s['frow'][:, :] - gR_sc

    Q = q_lat * AMc
    Bl = jnp.where(NM_ref[0] != 0, s['blat'][:, :], 0.0)
    return E, F, Q, Bl


_PKEYS = ['Wr', 'br', 'Wa1', 'ba1', 'Wa2', 'ba2', 'Wq1', 'bq1', 'Wq2', 'bq2',
          'Wqm1', 'bqm1', 'Wqm2', 'bqm2', 'Wb1', 'bb1', 'Wb2', 'bb2',
          'Wbm1', 'bbm1', 'Wbm2', 'bbm2', 'We1', 'be1', 'We2', 'be2']

_SCRATCH = [('e1f', (EDG, A)), ('rbf', (EDG, RES)),
            ('bdck', (NITER - 1, A, NN, NF)), ('gbd', (A, NN, NF)),
            ('grbf', (EDG, RES)), ('de', (A, NF)), ('gde', (A, NF)),
            ('am', (A, NF)), ('qdn', (A, NF)), ('gqdnr', (A, NF)),
            ('gamr', (A, NF)), ('gqdns', (A, NF)), ('gams', (A, NF)),
            ('gD', (A, NN)), ('blat', (A, NN)), ('fc', (A, NN)),
            ('dfc', (A, NN)), ('dinv', (A, NN)), ('dd', (A, NN)),
            ('frow', (A, 3)),
            ('ack', (NITER, A, NF)), ('qdck', (NITER, A, NF))]


def _body(R_ref, Z_ref, N_ref, AM_ref, NM_ref, emb_ref, *prefs):
    np_ = len(_PKEYS)
    piter_refs = prefs[:np_]
    U1_ref, c1_ref, U2_ref, c2_ref, U3_ref, c3_ref = prefs[np_:np_ + 6]
    E_ref, F_ref, Q_ref, Bl_ref = prefs[np_ + 6:np_ + 10]
    s = {k: r for (k, _), r in zip(_SCRATCH, prefs[np_ + 10:])}
    pref = dict(zip(_PKEYS, piter_refs))
    E, F, Q, Bl = _mol_body(R_ref, Z_ref, N_ref, AM_ref, NM_ref,
                            emb_ref[:, :], pref, U1_ref[:, :],
                            c1_ref[:, :], U2_ref[:, :], c2_ref[:, :],
                            U3_ref[:, :], c3_ref[:, :], s)
    E_ref[0] = jnp.zeros((1, 128), _F32) + E
    F_ref[0] = F
    Q_ref[0] = Q
    Bl_ref[0] = Bl


def _pack(params):
    nm_map = [('rbf', 0, 'Wr', 'br'), ('phi_a', 0, 'Wa1', 'ba1'),
              ('phi_a', 1, 'Wa2', 'ba2'), ('phi_q', 0, 'Wq1', 'bq1'),
              ('phi_q', 1, 'Wq2', 'bq2'), ('phi_qm', 0, 'Wqm1', 'bqm1'),
              ('phi_qm', 1, 'Wqm2', 'bqm2'), ('phi_b', 0, 'Wb1', 'bb1'),
              ('phi_b', 1, 'Wb2', 'bb2'), ('phi_bm', 0, 'Wbm1', 'bbm1'),
              ('phi_bm', 1, 'Wbm2', 'bbm2'), ('phi_e', 0, 'We1', 'be1'),
              ('phi_e', 1, 'We2', 'be2')]
    out = {}
    for name, li, wk, bk in nm_map:
        ws, bs = [], []
        for t in range(NITER):
            p = params['iters'][t][name]
            if isinstance(p, list):
                p = p[li]
            ws.append(p['w'])
            bs.append(p['b'].reshape(1, -1))
        out[wk] = jnp.stack(ws)
        out[bk] = jnp.stack(bs)
    emb = params['atom_emb']
    emb16 = jnp.zeros((16, NF), _F32).at[:emb.shape[0]].set(emb)
    at = params['atomic']
    return (out, emb16, at[0]['w'], at[0]['b'].reshape(1, -1),
            at[1]['w'], at[1]['b'].reshape(1, -1),
            at[2]['w'], at[2]['b'].reshape(1, -1))


def kernel(R, Z, N, AM, NM, params):
    B = R.shape[0]
    pk, emb16, U1, c1, U2, c2, U3, c3 = _pack(params)
    Zc = Z.astype(jnp.int32).reshape(B, A, 1)
    Nc = N.astype(jnp.int32)
    AMc = AM.reshape(B, A, 1)

    def cspec(x):
        nd = x.ndim
        return pl.BlockSpec(x.shape, lambda b, _n=nd: (0,) * _n)

    in_specs = [
        pl.BlockSpec((1, A, 3), lambda b: (b, 0, 0)),
        pl.BlockSpec((1, A, 1), lambda b: (b, 0, 0)),
        pl.BlockSpec((1, A, NN), lambda b: (b, 0, 0)),
        pl.BlockSpec((1, A, 1), lambda b: (b, 0, 0)),
        pl.BlockSpec((1, A, NN), lambda b: (b, 0, 0)),
        cspec(emb16),
    ]
    pargs = []
    for k in _PKEYS:
        pargs.append(pk[k])
        in_specs.append(cspec(pk[k]))
    for arr in (U1, c1, U2, c2, U3, c3):
        pargs.append(arr)
        in_specs.append(cspec(arr))

    out_specs = [
        pl.BlockSpec((1, 1, 128), lambda b: (b, 0, 0)),
        pl.BlockSpec((1, A, 3), lambda b: (b, 0, 0)),
        pl.BlockSpec((1, A, 1), lambda b: (b, 0, 0)),
        pl.BlockSpec((1, A, NN), lambda b: (b, 0, 0)),
    ]
    out_shape = [
        jax.ShapeDtypeStruct((B, 1, 128), _F32),
        jax.ShapeDtypeStruct((B, A, 3), _F32),
        jax.ShapeDtypeStruct((B, A, 1), _F32),
        jax.ShapeDtypeStruct((B, A, NN), _F32),
    ]
    Eo, Fo, Qo, Blo = pl.pallas_call(
        _body,
        grid=(B,),
        in_specs=in_specs,
        out_specs=out_specs,
        out_shape=out_shape,
        scratch_shapes=[pltpu.VMEM(shape, _F32) for _, shape in _SCRATCH],
        compiler_params=pltpu.CompilerParams(
            dimension_semantics=("arbitrary",),
        ),
    )(R, Zc, Nc, AMc, NM, emb16, *pargs)
    return (Eo[:, 0, :1], Fo, Qo[:, :, 0], Blo)


# speedup vs baseline: 4.4674x; 1.0002x over previous
import jax
import jax.numpy as jnp
from jax.experimental import pallas as pl
from jax.experimental.pallas import tpu as pltpu

A = 128
NN = 48
NF = 128
RES = 20
NITER = 3
CUTOFF = 5.0
PP = 9
EDG = A * NN
CH = 32
NC = A // CH
CE = CH * NN

_F32 = jnp.float32


def _sig(x):
    return jax.nn.sigmoid(x)


def _swish(x):
    return x * _sig(x)


def _dswish(u):
    s = _sig(u)
    return s * (1.0 + u * (1.0 - s))


_PREC = jax.lax.Precision.HIGHEST
_BF16 = jnp.bfloat16


def _mm(x, w):
    return jax.lax.dot_general(x.astype(_BF16), w.astype(_BF16),
                               (((1,), (0,)), ((), ())),
                               preferred_element_type=_F32)


def _mmT(x, w):
    return jax.lax.dot_general(x.astype(_BF16), w.astype(_BF16),
                               (((1,), (1,)), ((), ())),
                               preferred_element_type=_F32)


def _mmx(x, w):
    return jax.lax.dot_general(x, w, (((1,), (0,)), ((), ())),
                               precision=_PREC,
                               preferred_element_type=_F32)


def _scat(e1f, g):
    return jax.lax.dot_general(e1f, g, (((0,), (0,)), ((), ())),
                               precision=_PREC,
                               preferred_element_type=_F32)


def _atom_fwd(P, a, qd):
    ua = _mm(a, P['Wa1']) + P['ba1']
    am = _mm(_swish(ua), P['Wa2']) + P['ba2']
    uq = _mm(a, P['Wq1']) + P['bq1']
    q = _mm(_swish(uq), P['Wq2']) + P['bq2']
    uqm = _mm(a, P['Wqm1']) + P['bqm1']
    qm = _mm(_swish(uqm), P['Wqm2']) + P['bqm2']
    qdn = qd + q * qm
    ue = _mm(a, P['We1']) + P['be1']
    pe = _mm(_swish(ue), P['We2']) + P['be2']
    return dict(ua=ua, am=am, uq=uq, q=q, uqm=uqm, qm=qm, qdn=qdn,
                ue=ue, pe=pe)


def _chunk_fwd(P, s, t, c):
    r0 = c * CH
    e0 = c * CE
    rows = pl.ds(r0, CH)
    erows = pl.ds(e0, CE)
    ef = s['e1f'][erows, :]
    fc_c = s['fc'][rows, :]
    hr = _mm(s['rbf'][erows, :], P['Wr']) + P['br']
    rm = hr.reshape(CH, NN, NF) * fc_c[:, :, None]
    am_c = s['am'][rows, :]
    aj = _mmx(ef, s['am'][:, :]).reshape(CH, NN, NF)
    ms = am_c[:, None, :] * aj * rm
    msf = ms.reshape(CE, NF)
    ub = _mm(msf, P['Wb1']) + P['bb1']
    bij = (_mm(_swish(ub), P['Wb2']) + P['bb2']).reshape(CH, NN, 1)
    ubm = _mm(msf, P['Wbm1']) + P['bbm1']
    bm = (_mm(_swish(ubm), P['Wbm2']) + P['bbm2']).reshape(CH, NN, NF)
    if t > 0:
        bdn = s['bdck'][t - 1, rows] + bij * bm
    else:
        bdn = bij * bm
    qdn_c = s['qdn'][rows, :]
    qj = _mmx(ef, s['qdn'][:, :]).reshape(CH, NN, NF)
    qq = qdn_c[:, None, :] * qj
    de = jnp.sum(s['dinv'][rows, :][:, :, None] * (qq - bdn), axis=1)
    return dict(hr=hr, rm=rm, aj=aj, ub=ub, bij=bij, ubm=ubm, bm=bm,
                bdn=bdn, qj=qj, qq=qq, de=de, ef=ef, am_c=am_c,
                qdn_c=qdn_c, fc_c=fc_c, rows=rows, erows=erows)


def _mol_body(R_ref, Z_ref, N_ref, AM_ref, NM_ref, emb, pref,
              U1, c1, U2, c2, U3, c3, s):
    bnorm = jnp.sqrt(2.0 / CUTOFF)
    R = R_ref[0]
    AMc = AM_ref[0]

    def geo_body(c, _):
        rows = pl.ds(c * CH, CH)
        erows = pl.ds(c * CE, CE)
        nrows = N_ref[0, rows, :]
        nio = jax.lax.broadcasted_iota(jnp.int32, (CH, NN, A), 2)
        e1c = (nrows[:, :, None] == nio).astype(_F32).reshape(CE, A)
        s['e1f'][erows, :] = e1c
        Rj = _mmx(e1c, R).reshape(CH, NN, 3)
        V = Rj - R_ref[0, rows, :][:, None, :]
        Dsq = jnp.sum(V * V, axis=2)
        Ds = jnp.sqrt(jnp.maximum(Dsq, 1e-12))
        D = jnp.where(Dsq > 1e-9, Ds, 0.0) * NM_ref[0, rows, :]
        s['dd'][rows, :] = D
        pos = D > 0
        Dsafe = jnp.where(pos, D, 1.0)
        s['dinv'][rows, :] = jnp.where(pos, 1.0 / Dsafe, 0.0)
        d = D / CUTOFF
        d2 = d * d
        d4 = d2 * d2
        d8 = d4 * d4
        d9 = d8 * d
        d10 = d9 * d
        d11 = d10 * d
        inr = d < 1.0
        s['fc'][rows, :] = jnp.where(
            inr, 1.0 - 55.0 * d9 + 99.0 * d10 - 45.0 * d11, 0.0)
        s['dfc'][rows, :] = jnp.where(
            inr, (-495.0 * d8 + 990.0 * d9 - 495.0 * d10) / CUTOFF, 0.0)
        kio = jax.lax.broadcasted_iota(jnp.int32, (CH, NN, RES),
                                       2).astype(_F32) + 1.0
        Dx = D[:, :, None]
        arg = kio * (jnp.pi / CUTOFF) * Dx
        posx = Dx > 0
        sfx = jnp.where(posx, Dx, 1.0)
        s['rbf'][erows, :] = (bnorm * jnp.where(posx, jnp.sin(arg) / sfx,
                                                0.0)).reshape(CE, RES)
        return 0

    jax.lax.fori_loop(0, NC, geo_body, 0)

    zio = jax.lax.broadcasted_iota(jnp.int32, (A, 16), 1)
    zoh = (Z_ref[0] == zio).astype(_F32)
    a0 = _mmx(zoh, emb)
    s['blat'][:, :] = jnp.zeros((A, NN), _F32)

    a = a0
    qd = jnp.zeros((A, NF), _F32)
    q_lat = jnp.zeros((A, 1), _F32)
    for t in range(NITER):
        P = {k: r[t] for k, r in pref.items()}
        al = _atom_fwd(P, a, qd)
        s['ack'][t] = a
        s['qdck'][t] = qd
        s['am'][:, :] = al['am']
        s['qdn'][:, :] = al['qdn']

        def fbody(c, _, t=t, P=P):
            f = _chunk_fwd(P, s, t, c)
            if t < NITER - 1:
                s['bdck'][t, f['rows']] = f['bdn']
            s['de'][f['rows'], :] = f['de']
            s['blat'][f['rows'], :] = (s['blat'][f['rows'], :]
                                       + f['bij'][:, :, 0])
            return 0

        jax.lax.fori_loop(0, NC, fbody, 0)
        de = s['de'][:, :]
        a = a + al['pe'] * de
        qd = al['qdn']
        q_lat = q_lat + al['q']

    u1 = _mm(a, U1) + c1
    s1 = _swish(u1)
    u2 = _mm(s1, U2) + c2
    s2 = _swish(u2)
    Ei = (_mm(s2, U3) + c3) * AMc
    E = jnp.sum(Ei)

    g_s2 = _mmT(AMc, U3)
    g_s1 = _mmT(g_s2 * _dswish(u2), U2)
    ga = _mmT(g_s1 * _dswish(u1), U1)
    s['gbd'][:, :, :] = jnp.zeros((A, NN, NF), _F32)
    s['gD'][:, :] = jnp.zeros((A, NN), _F32)
    s['grbf'][:, :] = jnp.zeros((EDG, RES), _F32)

    gqd = jnp.zeros((A, NF), _F32)
    for t in range(NITER - 1, -1, -1):
        _P = {k: r[t] for k, r in pref.items()}
        P = _P
        al = _atom_fwd(P, s['ack'][t], s['qdck'][t])
        s['am'][:, :] = al['am']
        s['qdn'][:, :] = al['qdn']
        s['gde'][:, :] = ga * al['pe']
        s['gqdns'][:, :] = jnp.zeros((A, NF), _F32)
        s['gams'][:, :] = jnp.zeros((A, NF), _F32)

        def bbody(c, _, t=t, _P=_P):
            f = _chunk_fwd(_P, s, t, c)
            rows, erows = f['rows'], f['erows']
            dinv_c = s['dinv'][rows, :]
            g_de_c = s['gde'][rows, :]
            g_qq = dinv_c[:, :, None] * g_de_c[:, None, :]
            gbd_c = s['gbd'][rows] - g_qq
            s['gbd'][rows] = gbd_c
            g_Dinv = jnp.sum(g_de_c[:, None, :] * (f['qq'] - f['bdn']),
                             axis=2)
            gD_c = -(dinv_c * dinv_c) * g_Dinv
            g_bij = jnp.sum(gbd_c * f['bm'], axis=2).reshape(CE, 1)
            g_bm = (gbd_c * f['bij']).reshape(CE, NF)
            g_ms = _mmT(_mmT(g_bm, _P['Wbm2']) * _dswish(f['ubm']),
                        _P['Wbm1'])
            g_ms = g_ms + _mmT(_mmT(g_bij, _P['Wb2']) * _dswish(f['ub']),
                               _P['Wb1'])
            g_ms3 = g_ms.reshape(CH, NN, NF)
            s['gqdnr'][rows, :] = jnp.sum(g_qq * f['qj'], axis=1)
            s['gqdns'][:, :] = s['gqdns'][:, :] + _scat(
                f['ef'],
                (g_qq * f['qdn_c'][:, None, :]).reshape(CE, NF))
            am_b = f['am_c'][:, None, :]
            s['gamr'][rows, :] = jnp.sum(g_ms3 * f['aj'] * f['rm'], axis=1)
            s['gams'][:, :] = s['gams'][:, :] + _scat(
                f['ef'], (g_ms3 * am_b * f['rm']).reshape(CE, NF))
            g_rm = g_ms3 * am_b * f['aj']
            g_hr = (g_rm * f['fc_c'][:, :, None]).reshape(CE, NF)
            g_fc = jnp.sum(g_rm * f['hr'].reshape(CH, NN, NF), axis=2)
            s['gD'][rows, :] = (s['gD'][rows, :] + gD_c
                                + s['dfc'][rows, :] * g_fc)
            s['grbf'][erows, :] = (s['grbf'][erows, :]
                                   + _mmT(g_hr, _P['Wr']))
            s['de'][rows, :] = f['de']
            return 0

        jax.lax.fori_loop(0, NC, bbody, 0)
        de = s['de'][:, :]
        g_pe = ga * de
        ga = ga + _mmT(_mmT(g_pe, P['We2']) * _dswish(al['ue']), P['We1'])
        g_qdn = gqd + s['gqdnr'][:, :] + s['gqdns'][:, :]
        g_q = jnp.sum(g_qdn * al['qm'], axis=1, keepdims=True)
        g_qm = g_qdn * al['q']
        ga = ga + _mmT(_mmT(g_qm, P['Wqm2']) * _dswish(al['uqm']), P['Wqm1'])
        ga = ga + _mmT(_mmT(g_q, P['Wq2']) * _dswish(al['uq']), P['Wq1'])
        g_am = s['gamr'][:, :] + s['gams'][:, :]
        ga = ga + _mmT(_mmT(g_am, P['Wa2']) * _dswish(al['ua']), P['Wa1'])
        gqd = g_qdn

    def force_body(c, gR_sc):
        rows = pl.ds(c * CH, CH)
        erows = pl.ds(c * CE, CE)
        D = s['dd'][rows, :]
        Dx = D[:, :, None]
        posx = Dx > 0
        sfx = jnp.where(posx, Dx, 1.0)
        kio = jax.lax.broadcasted_iota(jnp.int32, (CH, NN, RES),
                                       2).astype(_F32) + 1.0
        arg = kio * (jnp.pi / CUTOFF) * Dx
        dbes = bnorm * jnp.where(
            posx,
            kio * (jnp.pi / CUTOFF) * jnp.cos(arg) / sfx
            - jnp.sin(arg) / (sfx * sfx), 0.0)
        gD = s['gD'][rows, :] + jnp.sum(
            s['grbf'][erows, :].reshape(CH, NN, RES) * dbes, axis=2)
        e1c = s['e1f'][erows, :]
        Rj = _mmx(e1c, R).reshape(CH, NN, 3)
        V = Rj - R_ref[0, rows, :][:, None, :]
        Dsq = jnp.sum(V * V, axis=2)
        Ds = jnp.sqrt(jnp.maximum(Dsq, 1e-12))
        gscale = jnp.where(Dsq > 1e-9, gD * NM_ref[0, rows, :] / Ds, 0.0)
        gV = gscale[:, :, None] * V
        s['frow'][rows, :] = jnp.sum(gV, axis=1)
        return gR_sc + _scat(e1c, gV.reshape(CE, 3))

    gR_sc = jax.lax.fori_loop(0, NC, force_body, jnp.zeros((A, 3), _F32))
    F = s['frow'][:, :] - gR_sc

    Q = q_lat * AMc
    Bl = jnp.where(NM_ref[0] != 0, s['blat'][:, :], 0.0)
    return E, F, Q, Bl


_PKEYS = ['Wr', 'br', 'Wa1', 'ba1', 'Wa2', 'ba2', 'Wq1', 'bq1', 'Wq2', 'bq2',
          'Wqm1', 'bqm1', 'Wqm2', 'bqm2', 'Wb1', 'bb1', 'Wb2', 'bb2',
          'Wbm1', 'bbm1', 'Wbm2', 'bbm2', 'We1', 'be1', 'We2', 'be2']

_SCRATCH = [('e1f', (EDG, A)), ('rbf', (EDG, RES)),
            ('bdck', (NITER - 1, A, NN, NF)), ('gbd', (A, NN, NF)),
            ('grbf', (EDG, RES)), ('de', (A, NF)), ('gde', (A, NF)),
            ('am', (A, NF)), ('qdn', (A, NF)), ('gqdnr', (A, NF)),
            ('gamr', (A, NF)), ('gqdns', (A, NF)), ('gams', (A, NF)),
            ('gD', (A, NN)), ('blat', (A, NN)), ('fc', (A, NN)),
            ('dfc', (A, NN)), ('dinv', (A, NN)), ('dd', (A, NN)),
            ('frow', (A, 3)),
            ('ack', (NITER, A, NF)), ('qdck', (NITER, A, NF))]


def _body(R_ref, Z_ref, N_ref, AM_ref, NM_ref, emb_ref, *prefs):
    np_ = len(_PKEYS)
    piter_refs = prefs[:np_]
    U1_ref, c1_ref, U2_ref, c2_ref, U3_ref, c3_ref = prefs[np_:np_ + 6]
    E_ref, F_ref, Q_ref, Bl_ref = prefs[np_ + 6:np_ + 10]
    s = {k: r for (k, _), r in zip(_SCRATCH, prefs[np_ + 10:])}
    pref = dict(zip(_PKEYS, piter_refs))
    E, F, Q, Bl = _mol_body(R_ref, Z_ref, N_ref, AM_ref, NM_ref,
                            emb_ref[:, :], pref, U1_ref[:, :],
                            c1_ref[:, :], U2_ref[:, :], c2_ref[:, :],
                            U3_ref[:, :], c3_ref[:, :], s)
    E_ref[0] = jnp.zeros((1, 128), _F32) + E
    F_ref[0] = F
    Q_ref[0] = Q
    Bl_ref[0] = Bl


def _pack(params):
    nm_map = [('rbf', 0, 'Wr', 'br'), ('phi_a', 0, 'Wa1', 'ba1'),
              ('phi_a', 1, 'Wa2', 'ba2'), ('phi_q', 0, 'Wq1', 'bq1'),
              ('phi_q', 1, 'Wq2', 'bq2'), ('phi_qm', 0, 'Wqm1', 'bqm1'),
              ('phi_qm', 1, 'Wqm2', 'bqm2'), ('phi_b', 0, 'Wb1', 'bb1'),
              ('phi_b', 1, 'Wb2', 'bb2'), ('phi_bm', 0, 'Wbm1', 'bbm1'),
              ('phi_bm', 1, 'Wbm2', 'bbm2'), ('phi_e', 0, 'We1', 'be1'),
              ('phi_e', 1, 'We2', 'be2')]
    out = {}
    for name, li, wk, bk in nm_map:
        ws, bs = [], []
        for t in range(NITER):
            p = params['iters'][t][name]
            if isinstance(p, list):
                p = p[li]
            ws.append(p['w'])
            bs.append(p['b'].reshape(1, -1))
        out[wk] = jnp.stack(ws)
        out[bk] = jnp.stack(bs)
    emb = params['atom_emb']
    emb16 = jnp.zeros((16, NF), _F32).at[:emb.shape[0]].set(emb)
    at = params['atomic']
    return (out, emb16, at[0]['w'], at[0]['b'].reshape(1, -1),
            at[1]['w'], at[1]['b'].reshape(1, -1),
            at[2]['w'], at[2]['b'].reshape(1, -1))


def kernel(R, Z, N, AM, NM, params):
    B = R.shape[0]
    pk, emb16, U1, c1, U2, c2, U3, c3 = _pack(params)
    Zc = Z.astype(jnp.int32).reshape(B, A, 1)
    Nc = N.astype(jnp.int32)
    AMc = AM.reshape(B, A, 1)

    def cspec(x):
        nd = x.ndim
        return pl.BlockSpec(x.shape, lambda b, _n=nd: (0,) * _n)

    in_specs = [
        pl.BlockSpec((1, A, 3), lambda b: (b, 0, 0)),
        pl.BlockSpec((1, A, 1), lambda b: (b, 0, 0)),
        pl.BlockSpec((1, A, NN), lambda b: (b, 0, 0)),
        pl.BlockSpec((1, A, 1), lambda b: (b, 0, 0)),
        pl.BlockSpec((1, A, NN), lambda b: (b, 0, 0)),
        cspec(emb16),
    ]
    pargs = []
    for k in _PKEYS:
        pargs.append(pk[k])
        in_specs.append(cspec(pk[k]))
    for arr in (U1, c1, U2, c2, U3, c3):
        pargs.append(arr)
        in_specs.append(cspec(arr))

    out_specs = [
        pl.BlockSpec((1, 1, 128), lambda b: (b, 0, 0)),
        pl.BlockSpec((1, A, 3), lambda b: (b, 0, 0)),
        pl.BlockSpec((1, A, 1), lambda b: (b, 0, 0)),
        pl.BlockSpec((1, A, NN), lambda b: (b, 0, 0)),
    ]
    out_shape = [
        jax.ShapeDtypeStruct((B, 1, 128), _F32),
        jax.ShapeDtypeStruct((B, A, 3), _F32),
        jax.ShapeDtypeStruct((B, A, 1), _F32),
        jax.ShapeDtypeStruct((B, A, NN), _F32),
    ]
    Eo, Fo, Qo, Blo = pl.pallas_call(
        _body,
        grid=(B,),
        in_specs=in_specs,
        out_specs=out_specs,
        out_shape=out_shape,
        scratch_shapes=[pltpu.VMEM(shape, _F32) for _, shape in _SCRATCH],
        compiler_params=pltpu.CompilerParams(
            dimension_semantics=("parallel",),
        ),
    )(R, Zc, Nc, AMc, NM, emb16, *pargs)
    return (Eo[:, 0, :1], Fo, Qo[:, :, 0], Blo)


# speedup vs baseline: 7.5917x; 1.6993x over previous
import jax
import jax.numpy as jnp
from jax.experimental import pallas as pl
from jax.experimental.pallas import tpu as pltpu

A = 128
NN = 48
NF = 128
RES = 20
NITER = 3
CUTOFF = 5.0
PP = 9
EDG = A * NN
CH = 32
NC = A // CH
CE = CH * NN

_F32 = jnp.float32


def _sig(x):
    return jax.nn.sigmoid(x)


def _swish(x):
    return x * _sig(x)


def _dswish(u):
    s = _sig(u)
    return s * (1.0 + u * (1.0 - s))


_PREC = jax.lax.Precision.HIGHEST
_BF16 = jnp.bfloat16


def _mm(x, w):
    return jax.lax.dot_general(x.astype(_BF16), w.astype(_BF16),
                               (((1,), (0,)), ((), ())),
                               preferred_element_type=_F32)


def _mmT(x, w):
    return jax.lax.dot_general(x.astype(_BF16), w.astype(_BF16),
                               (((1,), (1,)), ((), ())),
                               preferred_element_type=_F32)


def _hilo(x):
    hi = x.astype(_BF16)
    lo = (x - hi.astype(_F32)).astype(_BF16)
    return hi, lo


def _mmx(x, w):
    xb = x.astype(_BF16)
    hi, lo = _hilo(w)
    dims = (((1,), (0,)), ((), ()))
    return (jax.lax.dot_general(xb, hi, dims, preferred_element_type=_F32)
            + jax.lax.dot_general(xb, lo, dims,
                                  preferred_element_type=_F32))


def _scat(e1f, g):
    eb = e1f.astype(_BF16)
    hi, lo = _hilo(g)
    dims = (((0,), (0,)), ((), ()))
    return (jax.lax.dot_general(eb, hi, dims, preferred_element_type=_F32)
            + jax.lax.dot_general(eb, lo, dims,
                                  preferred_element_type=_F32))


def _atom_fwd(P, a, qd):
    ua = _mm(a, P['Wa1']) + P['ba1']
    am = _mm(_swish(ua), P['Wa2']) + P['ba2']
    uq = _mm(a, P['Wq1']) + P['bq1']
    q = _mm(_swish(uq), P['Wq2']) + P['bq2']
    uqm = _mm(a, P['Wqm1']) + P['bqm1']
    qm = _mm(_swish(uqm), P['Wqm2']) + P['bqm2']
    qdn = qd + q * qm
    ue = _mm(a, P['We1']) + P['be1']
    pe = _mm(_swish(ue), P['We2']) + P['be2']
    return dict(ua=ua, am=am, uq=uq, q=q, uqm=uqm, qm=qm, qdn=qdn,
                ue=ue, pe=pe)


def _chunk_fwd(P, s, t, c):
    r0 = c * CH
    e0 = c * CE
    rows = pl.ds(r0, CH)
    erows = pl.ds(e0, CE)
    ef = s['e1f'][erows, :]
    fc_c = s['fc'][rows, :]
    hr = _mm(s['rbf'][erows, :], P['Wr']) + P['br']
    rm = hr.reshape(CH, NN, NF) * fc_c[:, :, None]
    am_c = s['am'][rows, :]
    aj = _mmx(ef, s['am'][:, :]).reshape(CH, NN, NF)
    ms = am_c[:, None, :] * aj * rm
    msf = ms.reshape(CE, NF)
    ub = _mm(msf, P['Wb1']) + P['bb1']
    bij = (_mm(_swish(ub), P['Wb2']) + P['bb2']).reshape(CH, NN, 1)
    ubm = _mm(msf, P['Wbm1']) + P['bbm1']
    bm = (_mm(_swish(ubm), P['Wbm2']) + P['bbm2']).reshape(CH, NN, NF)
    if t > 0:
        bdn = s['bdck'][t - 1, rows] + bij * bm
    else:
        bdn = bij * bm
    qdn_c = s['qdn'][rows, :]
    qj = _mmx(ef, s['qdn'][:, :]).reshape(CH, NN, NF)
    qq = qdn_c[:, None, :] * qj
    de = jnp.sum(s['dinv'][rows, :][:, :, None] * (qq - bdn), axis=1)
    return dict(hr=hr, rm=rm, aj=aj, ub=ub, bij=bij, ubm=ubm, bm=bm,
                bdn=bdn, qj=qj, qq=qq, de=de, ef=ef, am_c=am_c,
                qdn_c=qdn_c, fc_c=fc_c, rows=rows, erows=erows)


def _mol_body(R_ref, Z_ref, N_ref, AM_ref, NM_ref, emb, pref,
              U1, c1, U2, c2, U3, c3, s):
    bnorm = jnp.sqrt(2.0 / CUTOFF)
    R = R_ref[0]
    AMc = AM_ref[0]

    def geo_body(c, _):
        rows = pl.ds(c * CH, CH)
        erows = pl.ds(c * CE, CE)
        nrows = N_ref[0, rows, :]
        nio = jax.lax.broadcasted_iota(jnp.int32, (CH, NN, A), 2)
        e1c = (nrows[:, :, None] == nio).astype(_F32).reshape(CE, A)
        s['e1f'][erows, :] = e1c
        Rj = _mmx(e1c, R).reshape(CH, NN, 3)
        V = Rj - R_ref[0, rows, :][:, None, :]
        Dsq = jnp.sum(V * V, axis=2)
        Ds = jnp.sqrt(jnp.maximum(Dsq, 1e-12))
        D = jnp.where(Dsq > 1e-9, Ds, 0.0) * NM_ref[0, rows, :]
        s['dd'][rows, :] = D
        pos = D > 0
        Dsafe = jnp.where(pos, D, 1.0)
        s['dinv'][rows, :] = jnp.where(pos, 1.0 / Dsafe, 0.0)
        d = D / CUTOFF
        d2 = d * d
        d4 = d2 * d2
        d8 = d4 * d4
        d9 = d8 * d
        d10 = d9 * d
        d11 = d10 * d
        inr = d < 1.0
        s['fc'][rows, :] = jnp.where(
            inr, 1.0 - 55.0 * d9 + 99.0 * d10 - 45.0 * d11, 0.0)
        s['dfc'][rows, :] = jnp.where(
            inr, (-495.0 * d8 + 990.0 * d9 - 495.0 * d10) / CUTOFF, 0.0)
        kio = jax.lax.broadcasted_iota(jnp.int32, (CH, NN, RES),
                                       2).astype(_F32) + 1.0
        Dx = D[:, :, None]
        arg = kio * (jnp.pi / CUTOFF) * Dx
        posx = Dx > 0
        sfx = jnp.where(posx, Dx, 1.0)
        s['rbf'][erows, :] = (bnorm * jnp.where(posx, jnp.sin(arg) / sfx,
                                                0.0)).reshape(CE, RES)
        return 0

    jax.lax.fori_loop(0, NC, geo_body, 0)

    zio = jax.lax.broadcasted_iota(jnp.int32, (A, 16), 1)
    zoh = (Z_ref[0] == zio).astype(_F32)
    a0 = _mmx(zoh, emb)
    s['blat'][:, :] = jnp.zeros((A, NN), _F32)

    a = a0
    qd = jnp.zeros((A, NF), _F32)
    q_lat = jnp.zeros((A, 1), _F32)
    for t in range(NITER):
        P = {k: r[t] for k, r in pref.items()}
        al = _atom_fwd(P, a, qd)
        s['ack'][t] = a
        s['qdck'][t] = qd
        s['am'][:, :] = al['am']
        s['qdn'][:, :] = al['qdn']

        def fbody(c, _, t=t, P=P):
            f = _chunk_fwd(P, s, t, c)
            if t < NITER - 1:
                s['bdck'][t, f['rows']] = f['bdn']
            s['de'][f['rows'], :] = f['de']
            s['blat'][f['rows'], :] = (s['blat'][f['rows'], :]
                                       + f['bij'][:, :, 0])
            return 0

        jax.lax.fori_loop(0, NC, fbody, 0)
        de = s['de'][:, :]
        a = a + al['pe'] * de
        qd = al['qdn']
        q_lat = q_lat + al['q']

    u1 = _mm(a, U1) + c1
    s1 = _swish(u1)
    u2 = _mm(s1, U2) + c2
    s2 = _swish(u2)
    Ei = (_mm(s2, U3) + c3) * AMc
    E = jnp.sum(Ei)

    g_s2 = _mmT(AMc, U3)
    g_s1 = _mmT(g_s2 * _dswish(u2), U2)
    ga = _mmT(g_s1 * _dswish(u1), U1)
    s['gbd'][:, :, :] = jnp.zeros((A, NN, NF), _F32)
    s['gD'][:, :] = jnp.zeros((A, NN), _F32)
    s['grbf'][:, :] = jnp.zeros((EDG, RES), _F32)

    gqd = jnp.zeros((A, NF), _F32)
    for t in range(NITER - 1, -1, -1):
        _P = {k: r[t] for k, r in pref.items()}
        P = _P
        al = _atom_fwd(P, s['ack'][t], s['qdck'][t])
        s['am'][:, :] = al['am']
        s['qdn'][:, :] = al['qdn']
        s['gde'][:, :] = ga * al['pe']
        s['gqdns'][:, :] = jnp.zeros((A, NF), _F32)
        s['gams'][:, :] = jnp.zeros((A, NF), _F32)

        def bbody(c, _, t=t, _P=_P):
            f = _chunk_fwd(_P, s, t, c)
            rows, erows = f['rows'], f['erows']
            dinv_c = s['dinv'][rows, :]
            g_de_c = s['gde'][rows, :]
            g_qq = dinv_c[:, :, None] * g_de_c[:, None, :]
            gbd_c = s['gbd'][rows] - g_qq
            s['gbd'][rows] = gbd_c
            g_Dinv = jnp.sum(g_de_c[:, None, :] * (f['qq'] - f['bdn']),
                             axis=2)
            gD_c = -(dinv_c * dinv_c) * g_Dinv
            g_bij = jnp.sum(gbd_c * f['bm'], axis=2).reshape(CE, 1)
            g_bm = (gbd_c * f['bij']).reshape(CE, NF)
            g_ms = _mmT(_mmT(g_bm, _P['Wbm2']) * _dswish(f['ubm']),
                        _P['Wbm1'])
            g_ms = g_ms + _mmT(_mmT(g_bij, _P['Wb2']) * _dswish(f['ub']),
                               _P['Wb1'])
            g_ms3 = g_ms.reshape(CH, NN, NF)
            s['gqdnr'][rows, :] = jnp.sum(g_qq * f['qj'], axis=1)
            s['gqdns'][:, :] = s['gqdns'][:, :] + _scat(
                f['ef'],
                (g_qq * f['qdn_c'][:, None, :]).reshape(CE, NF))
            am_b = f['am_c'][:, None, :]
            s['gamr'][rows, :] = jnp.sum(g_ms3 * f['aj'] * f['rm'], axis=1)
            s['gams'][:, :] = s['gams'][:, :] + _scat(
                f['ef'], (g_ms3 * am_b * f['rm']).reshape(CE, NF))
            g_rm = g_ms3 * am_b * f['aj']
            g_hr = (g_rm * f['fc_c'][:, :, None]).reshape(CE, NF)
            g_fc = jnp.sum(g_rm * f['hr'].reshape(CH, NN, NF), axis=2)
            s['gD'][rows, :] = (s['gD'][rows, :] + gD_c
                                + s['dfc'][rows, :] * g_fc)
            s['grbf'][erows, :] = (s['grbf'][erows, :]
                                   + _mmT(g_hr, _P['Wr']))
            s['de'][rows, :] = f['de']
            return 0

        jax.lax.fori_loop(0, NC, bbody, 0)
        de = s['de'][:, :]
        g_pe = ga * de
        ga = ga + _mmT(_mmT(g_pe, P['We2']) * _dswish(al['ue']), P['We1'])
        g_qdn = gqd + s['gqdnr'][:, :] + s['gqdns'][:, :]
        g_q = jnp.sum(g_qdn * al['qm'], axis=1, keepdims=True)
        g_qm = g_qdn * al['q']
        ga = ga + _mmT(_mmT(g_qm, P['Wqm2']) * _dswish(al['uqm']), P['Wqm1'])
        ga = ga + _mmT(_mmT(g_q, P['Wq2']) * _dswish(al['uq']), P['Wq1'])
        g_am = s['gamr'][:, :] + s['gams'][:, :]
        ga = ga + _mmT(_mmT(g_am, P['Wa2']) * _dswish(al['ua']), P['Wa1'])
        gqd = g_qdn

    def force_body(c, gR_sc):
        rows = pl.ds(c * CH, CH)
        erows = pl.ds(c * CE, CE)
        D = s['dd'][rows, :]
        Dx = D[:, :, None]
        posx = Dx > 0
        sfx = jnp.where(posx, Dx, 1.0)
        kio = jax.lax.broadcasted_iota(jnp.int32, (CH, NN, RES),
                                       2).astype(_F32) + 1.0
        arg = kio * (jnp.pi / CUTOFF) * Dx
        dbes = bnorm * jnp.where(
            posx,
            kio * (jnp.pi / CUTOFF) * jnp.cos(arg) / sfx
            - jnp.sin(arg) / (sfx * sfx), 0.0)
        gD = s['gD'][rows, :] + jnp.sum(
            s['grbf'][erows, :].reshape(CH, NN, RES) * dbes, axis=2)
        e1c = s['e1f'][erows, :]
        Rj = _mmx(e1c, R).reshape(CH, NN, 3)
        V = Rj - R_ref[0, rows, :][:, None, :]
        Dsq = jnp.sum(V * V, axis=2)
        Ds = jnp.sqrt(jnp.maximum(Dsq, 1e-12))
        gscale = jnp.where(Dsq > 1e-9, gD * NM_ref[0, rows, :] / Ds, 0.0)
        gV = gscale[:, :, None] * V
        s['frow'][rows, :] = jnp.sum(gV, axis=1)
        return gR_sc + _scat(e1c, gV.reshape(CE, 3))

    gR_sc = jax.lax.fori_loop(0, NC, force_body, jnp.zeros((A, 3), _F32))
    F = s['frow'][:, :] - gR_sc

    Q = q_lat * AMc
    Bl = jnp.where(NM_ref[0] != 0, s['blat'][:, :], 0.0)
    return E, F, Q, Bl


_PKEYS = ['Wr', 'br', 'Wa1', 'ba1', 'Wa2', 'ba2', 'Wq1', 'bq1', 'Wq2', 'bq2',
          'Wqm1', 'bqm1', 'Wqm2', 'bqm2', 'Wb1', 'bb1', 'Wb2', 'bb2',
          'Wbm1', 'bbm1', 'Wbm2', 'bbm2', 'We1', 'be1', 'We2', 'be2']

_SCRATCH = [('e1f', (EDG, A)), ('rbf', (EDG, RES)),
            ('bdck', (NITER - 1, A, NN, NF)), ('gbd', (A, NN, NF)),
            ('grbf', (EDG, RES)), ('de', (A, NF)), ('gde', (A, NF)),
            ('am', (A, NF)), ('qdn', (A, NF)), ('gqdnr', (A, NF)),
            ('gamr', (A, NF)), ('gqdns', (A, NF)), ('gams', (A, NF)),
            ('gD', (A, NN)), ('blat', (A, NN)), ('fc', (A, NN)),
            ('dfc', (A, NN)), ('dinv', (A, NN)), ('dd', (A, NN)),
            ('frow', (A, 3)),
            ('ack', (NITER, A, NF)), ('qdck', (NITER, A, NF))]


def _body(R_ref, Z_ref, N_ref, AM_ref, NM_ref, emb_ref, *prefs):
    np_ = len(_PKEYS)
    piter_refs = prefs[:np_]
    U1_ref, c1_ref, U2_ref, c2_ref, U3_ref, c3_ref = prefs[np_:np_ + 6]
    E_ref, F_ref, Q_ref, Bl_ref = prefs[np_ + 6:np_ + 10]
    s = {k: r for (k, _), r in zip(_SCRATCH, prefs[np_ + 10:])}
    pref = dict(zip(_PKEYS, piter_refs))
    E, F, Q, Bl = _mol_body(R_ref, Z_ref, N_ref, AM_ref, NM_ref,
                            emb_ref[:, :], pref, U1_ref[:, :],
                            c1_ref[:, :], U2_ref[:, :], c2_ref[:, :],
                            U3_ref[:, :], c3_ref[:, :], s)
    E_ref[0] = jnp.zeros((1, 128), _F32) + E
    F_ref[0] = F
    Q_ref[0] = Q
    Bl_ref[0] = Bl


def _pack(params):
    nm_map = [('rbf', 0, 'Wr', 'br'), ('phi_a', 0, 'Wa1', 'ba1'),
              ('phi_a', 1, 'Wa2', 'ba2'), ('phi_q', 0, 'Wq1', 'bq1'),
              ('phi_q', 1, 'Wq2', 'bq2'), ('phi_qm', 0, 'Wqm1', 'bqm1'),
              ('phi_qm', 1, 'Wqm2', 'bqm2'), ('phi_b', 0, 'Wb1', 'bb1'),
              ('phi_b', 1, 'Wb2', 'bb2'), ('phi_bm', 0, 'Wbm1', 'bbm1'),
              ('phi_bm', 1, 'Wbm2', 'bbm2'), ('phi_e', 0, 'We1', 'be1'),
              ('phi_e', 1, 'We2', 'be2')]
    out = {}
    for name, li, wk, bk in nm_map:
        ws, bs = [], []
        for t in range(NITER):
            p = params['iters'][t][name]
            if isinstance(p, list):
                p = p[li]
            ws.append(p['w'])
            bs.append(p['b'].reshape(1, -1))
        out[wk] = jnp.stack(ws)
        out[bk] = jnp.stack(bs)
    emb = params['atom_emb']
    emb16 = jnp.zeros((16, NF), _F32).at[:emb.shape[0]].set(emb)
    at = params['atomic']
    return (out, emb16, at[0]['w'], at[0]['b'].reshape(1, -1),
            at[1]['w'], at[1]['b'].reshape(1, -1),
            at[2]['w'], at[2]['b'].reshape(1, -1))


def kernel(R, Z, N, AM, NM, params):
    B = R.shape[0]
    pk, emb16, U1, c1, U2, c2, U3, c3 = _pack(params)
    Zc = Z.astype(jnp.int32).reshape(B, A, 1)
    Nc = N.astype(jnp.int32)
    AMc = AM.reshape(B, A, 1)

    def cspec(x):
        nd = x.ndim
        return pl.BlockSpec(x.shape, lambda b, _n=nd: (0,) * _n)

    in_specs = [
        pl.BlockSpec((1, A, 3), lambda b: (b, 0, 0)),
        pl.BlockSpec((1, A, 1), lambda b: (b, 0, 0)),
        pl.BlockSpec((1, A, NN), lambda b: (b, 0, 0)),
        pl.BlockSpec((1, A, 1), lambda b: (b, 0, 0)),
        pl.BlockSpec((1, A, NN), lambda b: (b, 0, 0)),
        cspec(emb16),
    ]
    pargs = []
    for k in _PKEYS:
        pargs.append(pk[k])
        in_specs.append(cspec(pk[k]))
    for arr in (U1, c1, U2, c2, U3, c3):
        pargs.append(arr)
        in_specs.append(cspec(arr))

    out_specs = [
        pl.BlockSpec((1, 1, 128), lambda b: (b, 0, 0)),
        pl.BlockSpec((1, A, 3), lambda b: (b, 0, 0)),
        pl.BlockSpec((1, A, 1), lambda b: (b, 0, 0)),
        pl.BlockSpec((1, A, NN), lambda b: (b, 0, 0)),
    ]
    out_shape = [
        jax.ShapeDtypeStruct((B, 1, 128), _F32),
        jax.ShapeDtypeStruct((B, A, 3), _F32),
        jax.ShapeDtypeStruct((B, A, 1), _F32),
        jax.ShapeDtypeStruct((B, A, NN), _F32),
    ]
    Eo, Fo, Qo, Blo = pl.pallas_call(
        _body,
        grid=(B,),
        in_specs=in_specs,
        out_specs=out_specs,
        out_shape=out_shape,
        scratch_shapes=[pltpu.VMEM(shape, _F32) for _, shape in _SCRATCH],
        compiler_params=pltpu.CompilerParams(
            dimension_semantics=("parallel",),
        ),
    )(R, Zc, Nc, AMc, NM, emb16, *pargs)
    return (Eo[:, 0, :1], Fo, Qo[:, :, 0], Blo)
